# Initial kernel scaffold; baseline (speedup 1.0000x reference)
import jax
import jax.numpy as jnp
from jax.experimental import pallas as pl


def kernel(x, edge_index, batch, W1, b1, W2, b2, W3, b3, Wp1, bp1, Wp2, bp2, Wp3, bp3, L1w, L1b, L2w, L2b, L3w, L3b):
    raise NotImplementedError("write your pallas kernel here")


# speedup vs baseline: 1.1033x; 1.1033x over previous
import jax
import jax.numpy as jnp
from jax.experimental import pallas as pl

_RATIO = 0.5
_TOPK_BI = 200


def _mm_body(x_ref, w_ref, o_ref):
    o_ref[...] = jnp.dot(x_ref[...], w_ref[...],
                         preferred_element_type=jnp.float32)


def _matmul(x, W):
    n, _ = x.shape
    h = W.shape[1]
    return pl.pallas_call(
        _mm_body,
        out_shape=jax.ShapeDtypeStruct((n, h), jnp.float32),
    )(x, W)


def _matvec_body(x_ref, w_ref, o_ref):
    o_ref[...] = jnp.sum(x_ref[...] * w_ref[...], axis=1, keepdims=True)


def _matvec(x, w_row):
    n = x.shape[0]
    return pl.pallas_call(
        _matvec_body,
        out_shape=jax.ShapeDtypeStruct((n, 1), jnp.float32),
    )(x, w_row)


def _combine_body(agg_ref, xw_ref, d2_ref, b_ref, nm_ref, o_ref):
    out = (agg_ref[...] + xw_ref[...] * d2_ref[...] + b_ref[...]) * nm_ref[...]
    o_ref[...] = jnp.maximum(out, 0.0)


def _combine_relu(agg, xw, d2, b, nmf):
    n, h = xw.shape
    return pl.pallas_call(
        _combine_body,
        out_shape=jax.ShapeDtypeStruct((n, h), jnp.float32),
    )(agg, xw, d2, b.reshape(1, h), nmf)


def _score_k_body(aggp_ref, sw_ref, d2_ref, bp_ref, nm_ref,
                  batch_row_ref, nm_row_ref, score_ref, k_ref):
    score = (aggp_ref[...] + sw_ref[...] * d2_ref[...]
             + bp_ref[...]) * nm_ref[...]
    score_ref[...] = score
    g = k_ref.shape[0]
    g_col = jax.lax.broadcasted_iota(jnp.int32, (g, 1), 0)
    seg = batch_row_ref[...] == g_col
    n_act = jnp.sum(jnp.where(seg, nm_row_ref[...], 0.0), axis=1,
                    keepdims=True)
    k_ref[...] = jnp.ceil(_RATIO * n_act)


def _score_and_k(aggp, sw, d2, bp, nmf, batch, num_graphs):
    n = sw.shape[0]
    return pl.pallas_call(
        _score_k_body,
        out_shape=(jax.ShapeDtypeStruct((n, 1), jnp.float32),
                   jax.ShapeDtypeStruct((num_graphs, 1), jnp.float32)),
    )(aggp, sw, d2, bp.reshape(1, 1), nmf,
      batch.reshape(1, n), nmf.reshape(1, n))


def _topk_body(sc_ref, bc_ref, ic_ref, nmc_ref, h_ref,
               sr_ref, br_ref, nmr_ref, kr_ref, xn_ref, keep_ref):
    neg = jnp.float32(-jnp.inf)
    n = sr_ref.shape[1]
    g = kr_ref.shape[1]
    sm_c = jnp.where(nmc_ref[...] > 0, sc_ref[...], neg)
    sm_r = jnp.where(nmr_ref[...] > 0, sr_ref[...], neg)
    same = br_ref[...] == bc_ref[...]
    jidx = jax.lax.broadcasted_iota(jnp.int32, (1, n), 1)
    before = (sm_r > sm_c) | ((sm_r == sm_c) & (jidx < ic_ref[...]))
    rank = jnp.sum(jnp.where(same & before, 1.0, 0.0), axis=1,
                   keepdims=True)
    g_row = jax.lax.broadcasted_iota(jnp.int32, (1, g), 1)
    kb = jnp.sum(jnp.where(bc_ref[...] == g_row, kr_ref[...], 0.0),
                 axis=1, keepdims=True)
    keep = ((nmc_ref[...] > 0) & (rank < kb)).astype(jnp.float32)
    xn_ref[...] = h_ref[...] * jnp.tanh(sc_ref[...]) * keep
    keep_ref[...] = keep


def _topk_mask(score, batch_col, idx_col, nmf, h, k_vec):
    n, hid = h.shape
    g = k_vec.shape[0]
    bi = _TOPK_BI
    grid = n // bi
    return pl.pallas_call(
        _topk_body,
        grid=(grid,),
        in_specs=[
            pl.BlockSpec((bi, 1), lambda i: (i, 0)),
            pl.BlockSpec((bi, 1), lambda i: (i, 0)),
            pl.BlockSpec((bi, 1), lambda i: (i, 0)),
            pl.BlockSpec((bi, 1), lambda i: (i, 0)),
            pl.BlockSpec((bi, hid), lambda i: (i, 0)),
            pl.BlockSpec((1, n), lambda i: (0, 0)),
            pl.BlockSpec((1, n), lambda i: (0, 0)),
            pl.BlockSpec((1, n), lambda i: (0, 0)),
            pl.BlockSpec((1, g), lambda i: (0, 0)),
        ],
        out_specs=(pl.BlockSpec((bi, hid), lambda i: (i, 0)),
                   pl.BlockSpec((bi, 1), lambda i: (i, 0))),
        out_shape=(jax.ShapeDtypeStruct((n, hid), jnp.float32),
                   jax.ShapeDtypeStruct((n, 1), jnp.float32)),
    )(score, batch_col, idx_col, nmf, h,
      score.reshape(1, n), batch_col.reshape(1, n), nmf.reshape(1, n),
      k_vec.reshape(1, g))


def _readout_body(x_ref, b_ref, keep_ref, mx_ref, av_ref):
    x = x_ref[...]
    bcol = b_ref[...]
    kept = keep_ref[...] > 0
    num_graphs = mx_ref.shape[0]

    def body(g, carry):
        mask = (bcol == g) & kept
        mx = jnp.max(jnp.where(mask, x, -jnp.inf), axis=0, keepdims=True)
        mx = jnp.where(mx == -jnp.inf, 0.0, mx)
        mf = mask.astype(jnp.float32)
        cnt = jnp.sum(mf)
        av = jnp.sum(x * mf, axis=0, keepdims=True) / jnp.maximum(cnt, 1.0)
        mx_ref[pl.ds(g, 1), :] = mx
        av_ref[pl.ds(g, 1), :] = av
        return carry

    jax.lax.fori_loop(0, num_graphs, body, 0)


def _readout(xn, batch_col, keep, num_graphs):
    n, hid = xn.shape
    mx, av = pl.pallas_call(
        _readout_body,
        out_shape=(jax.ShapeDtypeStruct((num_graphs, hid), jnp.float32),
                   jax.ShapeDtypeStruct((num_graphs, hid), jnp.float32)),
    )(xn, batch_col, keep)
    return jnp.concatenate([mx, av], axis=1)


def _head_body(z_ref, w1_ref, b1_ref, w2_ref, b2_ref, w3_ref, b3_ref,
               o_ref):
    z = z_ref[...]
    z = jnp.maximum(jnp.dot(z, w1_ref[...],
                            preferred_element_type=jnp.float32)
                    + b1_ref[...], 0.0)
    z = jnp.maximum(jnp.dot(z, w2_ref[...],
                            preferred_element_type=jnp.float32)
                    + b2_ref[...], 0.0)
    z = jnp.dot(z, w3_ref[...], preferred_element_type=jnp.float32) \
        + b3_ref[...]
    z = z - jnp.max(z, axis=1, keepdims=True)
    o_ref[...] = z - jnp.log(jnp.sum(jnp.exp(z), axis=1, keepdims=True))


def _head(z, L1w, L1b, L2w, L2b, L3w, L3b):
    g = z.shape[0]
    cls = L3w.shape[1]
    return pl.pallas_call(
        _head_body,
        out_shape=jax.ShapeDtypeStruct((g, cls), jnp.float32),
    )(z, L1w, L1b.reshape(1, -1), L2w, L2b.reshape(1, -1),
      L3w, L3b.reshape(1, -1))


def kernel(x, edge_index, batch, W1, b1, W2, b2, W3, b3,
           Wp1, bp1, Wp2, bp2, Wp3, bp3,
           L1w, L1b, L2w, L2b, L3w, L3b):
    n = x.shape[0]
    num_graphs = 64

    src, dst = edge_index[0], edge_index[1]
    batch = batch.astype(jnp.int32)
    batch_col = batch.reshape(n, 1)
    idx_col = jnp.arange(n, dtype=jnp.int32).reshape(n, 1)

    nmf = jnp.ones((n, 1), jnp.float32)
    ew = jnp.ones((src.shape[0],), jnp.float32)
    h = x

    layer_params = ((W1, b1, Wp1, bp1), (W2, b2, Wp2, bp2),
                    (W3, b3, Wp3, bp3))
    reads = []
    for (W, b, Wp, bp) in layer_params:
        deg = jax.ops.segment_sum(ew, dst, num_segments=n) + nmf[:, 0]
        dis = jnp.where(deg > 0, jax.lax.rsqrt(deg), 0.0)
        d2 = (nmf[:, 0] * dis * dis).reshape(n, 1)
        coef = ew * dis[src] * dis[dst]

        xw = _matmul(h, W)
        agg = jax.ops.segment_sum(xw[src] * coef[:, None], dst,
                                  num_segments=n)
        h = _combine_relu(agg, xw, d2, b, nmf)

        sw = _matvec(h, Wp.reshape(1, -1))
        aggp = jax.ops.segment_sum(sw[:, 0][src] * coef, dst,
                                   num_segments=n).reshape(n, 1)
        score, k_vec = _score_and_k(aggp, sw, d2, bp, nmf, batch,
                                    num_graphs)

        h, keep = _topk_mask(score, batch_col, idx_col, nmf, h, k_vec)

        reads.append(_readout(h, batch_col, keep, num_graphs))

        keep1 = keep[:, 0]
        ew = ew * keep1[src] * keep1[dst]
        nmf = keep

    z = reads[0] + reads[1] + reads[2]
    return _head(z, L1w, L1b, L2w, L2b, L3w, L3b)


# speedup vs baseline: 1.1034x; 1.0000x over previous
import jax
import jax.numpy as jnp
from jax.experimental import pallas as pl

_RATIO = 0.5
_TOPK_BI = 400


def _mm_body(x_ref, w_ref, o_ref):
    o_ref[...] = jnp.dot(x_ref[...], w_ref[...],
                         preferred_element_type=jnp.float32)


def _matmul(x, W):
    n, _ = x.shape
    h = W.shape[1]
    return pl.pallas_call(
        _mm_body,
        out_shape=jax.ShapeDtypeStruct((n, h), jnp.float32),
    )(x, W)


def _matvec_body(x_ref, w_ref, o_ref):
    o_ref[...] = jnp.sum(x_ref[...] * w_ref[...], axis=1, keepdims=True)


def _matvec(x, w_row):
    n = x.shape[0]
    return pl.pallas_call(
        _matvec_body,
        out_shape=jax.ShapeDtypeStruct((n, 1), jnp.float32),
    )(x, w_row)


def _combine_body(agg_ref, xw_ref, d2_ref, b_ref, nm_ref, o_ref):
    out = (agg_ref[...] + xw_ref[...] * d2_ref[...] + b_ref[...]) * nm_ref[...]
    o_ref[...] = jnp.maximum(out, 0.0)


def _combine_relu(agg, xw, d2, b, nmf):
    n, h = xw.shape
    return pl.pallas_call(
        _combine_body,
        out_shape=jax.ShapeDtypeStruct((n, h), jnp.float32),
    )(agg, xw, d2, b.reshape(1, h), nmf)


def _score_k_body(aggp_ref, sw_ref, d2_ref, bp_ref, nm_ref,
                  batch_row_ref, nm_row_ref, score_ref, k_ref):
    score = (aggp_ref[...] + sw_ref[...] * d2_ref[...]
             + bp_ref[...]) * nm_ref[...]
    score_ref[...] = score
    g = k_ref.shape[0]
    g_col = jax.lax.broadcasted_iota(jnp.int32, (g, 1), 0)
    seg = batch_row_ref[...] == g_col
    n_act = jnp.sum(jnp.where(seg, nm_row_ref[...], 0.0), axis=1,
                    keepdims=True)
    k_ref[...] = jnp.ceil(_RATIO * n_act)


def _score_and_k(aggp, sw, d2, bp, nmf, batch, num_graphs):
    n = sw.shape[0]
    return pl.pallas_call(
        _score_k_body,
        out_shape=(jax.ShapeDtypeStruct((n, 1), jnp.float32),
                   jax.ShapeDtypeStruct((num_graphs, 1), jnp.float32)),
    )(aggp, sw, d2, bp.reshape(1, 1), nmf,
      batch.reshape(1, n), nmf.reshape(1, n))


def _topk_body(sc_ref, bc_ref, ic_ref, nmc_ref, h_ref,
               sr_ref, br_ref, nmr_ref, kr_ref, xn_ref, keep_ref):
    neg = jnp.float32(-jnp.inf)
    n = sr_ref.shape[1]
    g = kr_ref.shape[1]
    sm_c = jnp.where(nmc_ref[...] > 0, sc_ref[...], neg)
    sm_r = jnp.where(nmr_ref[...] > 0, sr_ref[...], neg)
    same = br_ref[...] == bc_ref[...]
    jidx = jax.lax.broadcasted_iota(jnp.int32, (1, n), 1)
    before = (sm_r > sm_c) | ((sm_r == sm_c) & (jidx < ic_ref[...]))
    rank = jnp.sum(jnp.where(same & before, 1.0, 0.0), axis=1,
                   keepdims=True)
    g_row = jax.lax.broadcasted_iota(jnp.int32, (1, g), 1)
    kb = jnp.sum(jnp.where(bc_ref[...] == g_row, kr_ref[...], 0.0),
                 axis=1, keepdims=True)
    keep = ((nmc_ref[...] > 0) & (rank < kb)).astype(jnp.float32)
    xn_ref[...] = h_ref[...] * jnp.tanh(sc_ref[...]) * keep
    keep_ref[...] = keep


def _topk_mask(score, batch_col, idx_col, nmf, h, k_vec):
    n, hid = h.shape
    g = k_vec.shape[0]
    bi = _TOPK_BI
    grid = n // bi
    return pl.pallas_call(
        _topk_body,
        grid=(grid,),
        in_specs=[
            pl.BlockSpec((bi, 1), lambda i: (i, 0)),
            pl.BlockSpec((bi, 1), lambda i: (i, 0)),
            pl.BlockSpec((bi, 1), lambda i: (i, 0)),
            pl.BlockSpec((bi, 1), lambda i: (i, 0)),
            pl.BlockSpec((bi, hid), lambda i: (i, 0)),
            pl.BlockSpec((1, n), lambda i: (0, 0)),
            pl.BlockSpec((1, n), lambda i: (0, 0)),
            pl.BlockSpec((1, n), lambda i: (0, 0)),
            pl.BlockSpec((1, g), lambda i: (0, 0)),
        ],
        out_specs=(pl.BlockSpec((bi, hid), lambda i: (i, 0)),
                   pl.BlockSpec((bi, 1), lambda i: (i, 0))),
        out_shape=(jax.ShapeDtypeStruct((n, hid), jnp.float32),
                   jax.ShapeDtypeStruct((n, 1), jnp.float32)),
    )(score, batch_col, idx_col, nmf, h,
      score.reshape(1, n), batch_col.reshape(1, n), nmf.reshape(1, n),
      k_vec.reshape(1, g))


def _readout_body(x_ref, b_ref, keep_ref, mx_ref, av_ref):
    x = x_ref[...]
    bcol = b_ref[...]
    kept = keep_ref[...] > 0
    num_graphs = mx_ref.shape[0]

    def body(g, carry):
        mask = (bcol == g) & kept
        mx = jnp.max(jnp.where(mask, x, -jnp.inf), axis=0, keepdims=True)
        mx = jnp.where(mx == -jnp.inf, 0.0, mx)
        mf = mask.astype(jnp.float32)
        cnt = jnp.sum(mf)
        av = jnp.sum(x * mf, axis=0, keepdims=True) / jnp.maximum(cnt, 1.0)
        mx_ref[pl.ds(g, 1), :] = mx
        av_ref[pl.ds(g, 1), :] = av
        return carry

    jax.lax.fori_loop(0, num_graphs, body, 0)


def _readout(xn, batch_col, keep, num_graphs):
    n, hid = xn.shape
    mx, av = pl.pallas_call(
        _readout_body,
        out_shape=(jax.ShapeDtypeStruct((num_graphs, hid), jnp.float32),
                   jax.ShapeDtypeStruct((num_graphs, hid), jnp.float32)),
    )(xn, batch_col, keep)
    return jnp.concatenate([mx, av], axis=1)


def _head_body(z_ref, w1_ref, b1_ref, w2_ref, b2_ref, w3_ref, b3_ref,
               o_ref):
    z = z_ref[...]
    z = jnp.maximum(jnp.dot(z, w1_ref[...],
                            preferred_element_type=jnp.float32)
                    + b1_ref[...], 0.0)
    z = jnp.maximum(jnp.dot(z, w2_ref[...],
                            preferred_element_type=jnp.float32)
                    + b2_ref[...], 0.0)
    z = jnp.dot(z, w3_ref[...], preferred_element_type=jnp.float32) \
        + b3_ref[...]
    z = z - jnp.max(z, axis=1, keepdims=True)
    o_ref[...] = z - jnp.log(jnp.sum(jnp.exp(z), axis=1, keepdims=True))


def _head(z, L1w, L1b, L2w, L2b, L3w, L3b):
    g = z.shape[0]
    cls = L3w.shape[1]
    return pl.pallas_call(
        _head_body,
        out_shape=jax.ShapeDtypeStruct((g, cls), jnp.float32),
    )(z, L1w, L1b.reshape(1, -1), L2w, L2b.reshape(1, -1),
      L3w, L3b.reshape(1, -1))


def kernel(x, edge_index, batch, W1, b1, W2, b2, W3, b3,
           Wp1, bp1, Wp2, bp2, Wp3, bp3,
           L1w, L1b, L2w, L2b, L3w, L3b):
    n = x.shape[0]
    num_graphs = 64

    src, dst = edge_index[0], edge_index[1]
    batch = batch.astype(jnp.int32)
    batch_col = batch.reshape(n, 1)
    idx_col = jnp.arange(n, dtype=jnp.int32).reshape(n, 1)

    nmf = jnp.ones((n, 1), jnp.float32)
    ew = jnp.ones((src.shape[0],), jnp.float32)
    h = x

    layer_params = ((W1, b1, Wp1, bp1), (W2, b2, Wp2, bp2),
                    (W3, b3, Wp3, bp3))
    reads = []
    for (W, b, Wp, bp) in layer_params:
        deg = jax.ops.segment_sum(ew, dst, num_segments=n) + nmf[:, 0]
        dis = jnp.where(deg > 0, jax.lax.rsqrt(deg), 0.0)
        d2 = (nmf[:, 0] * dis * dis).reshape(n, 1)
        coef = ew * dis[src] * dis[dst]

        xw = _matmul(h, W)
        agg = jax.ops.segment_sum(xw[src] * coef[:, None], dst,
                                  num_segments=n)
        h = _combine_relu(agg, xw, d2, b, nmf)

        sw = _matvec(h, Wp.reshape(1, -1))
        aggp = jax.ops.segment_sum(sw[:, 0][src] * coef, dst,
                                   num_segments=n).reshape(n, 1)
        score, k_vec = _score_and_k(aggp, sw, d2, bp, nmf, batch,
                                    num_graphs)

        h, keep = _topk_mask(score, batch_col, idx_col, nmf, h, k_vec)

        reads.append(_readout(h, batch_col, keep, num_graphs))

        keep1 = keep[:, 0]
        ew = ew * keep1[src] * keep1[dst]
        nmf = keep

    z = reads[0] + reads[1] + reads[2]
    return _head(z, L1w, L1b, L2w, L2b, L3w, L3b)
